# baseline (speedup 1.0000x reference)
import functools
import math

import jax
import jax.numpy as jnp
from jax import lax
from jax.experimental import pallas as pl
from jax.experimental.pallas import tpu as pltpu
from jax.experimental.pallas import tpu_sc as plsc

D_MODEL = 64
SCALE = math.sqrt(D_MODEL)
NUM_CORES = 2
NUM_SUBCORES = 16
NW = NUM_CORES * NUM_SUBCORES
CHUNK = 128


def _make_embed(B, V):
    b_per_w = B // NW
    n_chunks = b_per_w // CHUNK
    mesh = plsc.VectorSubcoreMesh(core_axis_name="c", subcore_axis_name="s")

    @functools.partial(
        pl.kernel,
        mesh=mesh,
        compiler_params=pltpu.CompilerParams(use_tc_tiling_on_sc=False),
        out_type=jax.ShapeDtypeStruct((B, D_MODEL), jnp.float32),
        scratch_types=[
            pltpu.VMEM((n_chunks, CHUNK), jnp.int32),
            pltpu.VMEM((CHUNK, D_MODEL), jnp.float32),
            pltpu.SemaphoreType.DMA,
        ],
    )
    def k(table_hbm, idx_hbm, out_hbm, idx_v, rows_v, gsem):
        wid = lax.axis_index("s") * NUM_CORES + lax.axis_index("c")
        base = wid * b_per_w
        pltpu.sync_copy(idx_hbm.at[wid], idx_v)

        def chunk_body(j, _):
            pltpu.async_copy(table_hbm.at[idx_v.at[j]], rows_v, gsem).wait()

            def row_body(r, _):
                for q in range(D_MODEL // 16):
                    s = rows_v[r, pl.ds(q * 16, 16)]
                    rows_v[r, pl.ds(q * 16, 16)] = s * SCALE
                return 0

            lax.fori_loop(0, CHUNK, row_body, 0, unroll=4)
            pltpu.sync_copy(rows_v, out_hbm.at[pl.ds(base + j * CHUNK, CHUNK)])
            return 0

        lax.fori_loop(0, n_chunks, chunk_body, 0)

    return k


def kernel(x, table):
    B = x.size
    V = table.shape[0]
    idx = x.reshape(NW, B // NW // CHUNK, CHUNK).astype(jnp.int32)
    out = _make_embed(B, V)(table, idx)
    return out.reshape(*x.shape, D_MODEL)


# speedup vs baseline: 1.1587x; 1.1587x over previous
import functools
import math

import jax
import jax.numpy as jnp
from jax import lax
from jax.experimental import pallas as pl
from jax.experimental.pallas import tpu as pltpu
from jax.experimental.pallas import tpu_sc as plsc

D_MODEL = 64
SCALE = math.sqrt(D_MODEL)
NUM_CORES = 2
NUM_SUBCORES = 16
NW = NUM_CORES * NUM_SUBCORES
CHUNK = 128
NBUF = 4


def _make_embed(B, V):
    b_per_w = B // NW
    n_chunks = b_per_w // CHUNK
    n_outer = n_chunks // NBUF
    mesh = plsc.VectorSubcoreMesh(core_axis_name="c", subcore_axis_name="s")

    scratch = (
        [pltpu.VMEM((n_chunks, CHUNK), jnp.int32)]
        + [pltpu.VMEM((CHUNK, D_MODEL), jnp.float32) for _ in range(NBUF)]
        + [pltpu.SemaphoreType.DMA for _ in range(2 * NBUF)]
    )

    @functools.partial(
        pl.kernel,
        mesh=mesh,
        compiler_params=pltpu.CompilerParams(use_tc_tiling_on_sc=False),
        out_type=jax.ShapeDtypeStruct((B, D_MODEL), jnp.float32),
        scratch_types=scratch,
    )
    def k(table_hbm, idx_hbm, out_hbm, idx_v, *rest):
        rows = rest[:NBUF]
        gsem = rest[NBUF:2 * NBUF]
        osem = rest[2 * NBUF:3 * NBUF]
        wid = lax.axis_index("s") * NUM_CORES + lax.axis_index("c")
        base = wid * b_per_w
        pltpu.sync_copy(idx_hbm.at[wid], idx_v)
        for b in range(NBUF):
            pltpu.async_copy(table_hbm.at[idx_v.at[b]], rows[b], gsem[b])

        def outer(t, _):
            for b in range(NBUF):
                j = t * NBUF + b
                rb = (b + 2) % NBUF
                jj = j + 2

                @pl.when(jnp.logical_and(jj >= NBUF, jj < n_chunks))
                def _():
                    pltpu.make_async_copy(
                        rows[rb], out_hbm.at[pl.ds(base, CHUNK)], osem[rb]
                    ).wait()
                    pltpu.async_copy(table_hbm.at[idx_v.at[jj]], rows[rb], gsem[rb])

                pltpu.make_async_copy(
                    table_hbm.at[idx_v.at[j]], rows[b], gsem[b]
                ).wait()

                def row_body(r, _, b=b):
                    for q in range(D_MODEL // 16):
                        s = rows[b][r, pl.ds(q * 16, 16)]
                        rows[b][r, pl.ds(q * 16, 16)] = s * SCALE
                    return 0

                lax.fori_loop(0, CHUNK, row_body, 0, unroll=4)
                pltpu.async_copy(
                    rows[b], out_hbm.at[pl.ds(base + j * CHUNK, CHUNK)], osem[b]
                )
            return 0

        lax.fori_loop(0, n_outer, outer, 0)
        for b in range(NBUF):
            pltpu.make_async_copy(
                rows[b], out_hbm.at[pl.ds(base, CHUNK)], osem[b]
            ).wait()

    return k


def kernel(x, table):
    B = x.size
    V = table.shape[0]
    idx = x.reshape(NW, B // NW // CHUNK, CHUNK).astype(jnp.int32)
    out = _make_embed(B, V)(table, idx)
    return out.reshape(*x.shape, D_MODEL)
